# speedup vs baseline: 2.4732x; 2.4732x over previous
import jax
import jax.numpy as jnp
from jax.experimental import pallas as pl
from jax.experimental.pallas import tpu as pltpu

SEQ, ENVS, N_NODES, D_IN = 4, 8, 10000, 128
HID = 128
B = SEQ * ENVS


def _fused_kernel(x_ref, w0_ref, b0_ref, w1_ref, b1_ref, w2_ref, b2_ref,
                  wa_ref, ba_ref, wc_ref, bc_ref, logits_ref, values_ref):
    x = x_ref[0]
    inv_n = jnp.float32(1.0 / N_NODES)

    def layer(x, w_ref, b_ref):
        m = jnp.sum(x, axis=0, keepdims=True) * inv_n
        c = jnp.dot(m, w_ref[HID:, :],
                    preferred_element_type=jnp.float32) + b_ref[:]
        out = jnp.dot(x, w_ref[:HID, :],
                      preferred_element_type=jnp.float32) + c
        return jnp.maximum(out, 0.0)

    h0 = layer(x, w0_ref, b0_ref)
    h1 = layer(h0, w1_ref, b1_ref) + h0
    h2 = layer(h1, w2_ref, b2_ref) + h1

    m = jnp.sum(h2, axis=0, keepdims=True) * inv_n
    logits = jnp.dot(h2, wa_ref[:HID, :],
                     preferred_element_type=jnp.float32)
    logits = logits + (jnp.dot(m, wa_ref[HID:, :],
                               preferred_element_type=jnp.float32) + ba_ref[:])
    logits_ref[0] = logits

    val = jnp.dot(m, wc_ref[:HID, :] + wc_ref[HID:, :],
                  preferred_element_type=jnp.float32) + bc_ref[:]
    values_ref[0] = val


def kernel(states, W0, b0, W1, b1, W2, b2, Wa, ba, Wc, bc):
    x = states.reshape(B, N_NODES, D_IN)
    b0r = b0.reshape(1, HID)
    b1r = b1.reshape(1, HID)
    b2r = b2.reshape(1, HID)
    bar = ba.reshape(1, 1)
    bcr = bc.reshape(1, 1)

    full = lambda shape: pl.BlockSpec(shape, lambda i: (0,) * len(shape))

    logits, values = pl.pallas_call(
        _fused_kernel,
        grid=(B,),
        in_specs=[
            pl.BlockSpec((1, N_NODES, D_IN), lambda i: (i, 0, 0)),
            full((2 * D_IN, HID)), full((1, HID)),
            full((2 * HID, HID)), full((1, HID)),
            full((2 * HID, HID)), full((1, HID)),
            full((2 * HID, 1)), full((1, 1)),
            full((2 * HID, 1)), full((1, 1)),
        ],
        out_specs=[
            pl.BlockSpec((1, N_NODES, 1), lambda i: (i, 0, 0)),
            pl.BlockSpec((1, 1, 1), lambda i: (i, 0, 0)),
        ],
        out_shape=[
            jax.ShapeDtypeStruct((B, N_NODES, 1), jnp.float32),
            jax.ShapeDtypeStruct((B, 1, 1), jnp.float32),
        ],
        compiler_params=pltpu.CompilerParams(
            dimension_semantics=("arbitrary",),
        ),
    )(x, W0, b0r, W1, b1r, W2, b2r, Wa, bar, Wc, bcr)

    return (logits.reshape(SEQ, ENVS, N_NODES), values.reshape(SEQ, ENVS))


# speedup vs baseline: 2.6104x; 1.0555x over previous
import jax
import jax.numpy as jnp
from jax.experimental import pallas as pl
from jax.experimental.pallas import tpu as pltpu

SEQ, ENVS, N_NODES, D_IN = 4, 8, 10000, 128
HID = 128
B = SEQ * ENVS


def _colsum(x):
    part = jnp.sum(x.reshape(10, N_NODES // 10, x.shape[-1]), axis=0)
    return jnp.sum(part, axis=0, keepdims=True)


def _fused_kernel(x_ref, w0_ref, b0_ref, w1_ref, b1_ref, w2_ref, b2_ref,
                  wa_ref, ba_ref, wc_ref, bc_ref, logits_ref, values_ref):
    x = x_ref[0]
    inv_n = jnp.float32(1.0 / N_NODES)

    def layer(x, w_ref, b_ref, skip=None):
        m = _colsum(x) * inv_n
        c = jnp.dot(m, w_ref[HID:, :],
                    preferred_element_type=jnp.float32) + b_ref[:]
        out = jnp.dot(x, w_ref[:HID, :],
                      preferred_element_type=jnp.float32) + c
        out = jnp.maximum(out, 0.0)
        if skip is not None:
            out = out + skip
        return out

    h0 = layer(x, w0_ref, b0_ref)
    h1 = layer(h0, w1_ref, b1_ref, skip=h0)
    h2 = layer(h1, w2_ref, b2_ref, skip=h1)

    m = _colsum(h2) * inv_n
    logits = jnp.dot(h2, wa_ref[:HID, :],
                     preferred_element_type=jnp.float32)
    logits = logits + (jnp.dot(m, wa_ref[HID:, :],
                               preferred_element_type=jnp.float32) + ba_ref[:])
    logits_ref[0] = logits

    val = jnp.dot(m, wc_ref[:HID, :] + wc_ref[HID:, :],
                  preferred_element_type=jnp.float32) + bc_ref[:]
    values_ref[0] = val


def kernel(states, W0, b0, W1, b1, W2, b2, Wa, ba, Wc, bc):
    x = states.reshape(B, N_NODES, D_IN)
    b0r = b0.reshape(1, HID)
    b1r = b1.reshape(1, HID)
    b2r = b2.reshape(1, HID)
    bar = ba.reshape(1, 1)
    bcr = bc.reshape(1, 1)

    full = lambda shape: pl.BlockSpec(shape, lambda i: (0,) * len(shape))

    logits, values = pl.pallas_call(
        _fused_kernel,
        grid=(B,),
        in_specs=[
            pl.BlockSpec((1, N_NODES, D_IN), lambda i: (i, 0, 0)),
            full((2 * D_IN, HID)), full((1, HID)),
            full((2 * HID, HID)), full((1, HID)),
            full((2 * HID, HID)), full((1, HID)),
            full((2 * HID, 1)), full((1, 1)),
            full((2 * HID, 1)), full((1, 1)),
        ],
        out_specs=[
            pl.BlockSpec((1, N_NODES, 1), lambda i: (i, 0, 0)),
            pl.BlockSpec((1, 1, 1), lambda i: (i, 0, 0)),
        ],
        out_shape=[
            jax.ShapeDtypeStruct((B, N_NODES, 1), jnp.float32),
            jax.ShapeDtypeStruct((B, 1, 1), jnp.float32),
        ],
        compiler_params=pltpu.CompilerParams(
            dimension_semantics=("arbitrary",),
        ),
    )(x, W0, b0r, W1, b1r, W2, b2r, Wa, bar, Wc, bcr)

    return (logits.reshape(SEQ, ENVS, N_NODES), values.reshape(SEQ, ENVS))


# speedup vs baseline: 3.0599x; 1.1722x over previous
import jax
import jax.numpy as jnp
from jax.experimental import pallas as pl
from jax.experimental.pallas import tpu as pltpu

SEQ, ENVS, N_NODES, D_IN = 4, 8, 10000, 128
HID = 128
B = SEQ * ENVS


def _colsum(x):
    part = jnp.sum(x.reshape(10, N_NODES // 10, x.shape[-1]), axis=0)
    return jnp.sum(part, axis=0, keepdims=True)


def _fused_kernel(x_ref, w0_ref, b0_ref, w1_ref, b1_ref, w2_ref, b2_ref,
                  wht_ref, whb_ref, bh_ref, logits_ref, values_ref):
    x = x_ref[0]
    inv_n = jnp.float32(1.0 / N_NODES)

    def layer(x, w_ref, b_ref, skip=None):
        m = _colsum(x) * inv_n
        c = jnp.dot(m, w_ref[HID:, :],
                    preferred_element_type=jnp.float32) + b_ref[:]
        out = jnp.dot(x, w_ref[:HID, :],
                      preferred_element_type=jnp.float32) + c
        out = jnp.maximum(out, 0.0)
        if skip is not None:
            out = out + skip
        return out

    h0 = layer(x, w0_ref, b0_ref)
    h1 = layer(h0, w1_ref, b1_ref, skip=h0)
    h2 = layer(h1, w2_ref, b2_ref, skip=h1)

    m = _colsum(h2) * inv_n
    heads = jnp.dot(h2, wht_ref[:],
                    preferred_element_type=jnp.float32)
    hconst = jnp.dot(m, whb_ref[:],
                     preferred_element_type=jnp.float32) + bh_ref[:]

    logits_ref[0] = heads[:, 0:1] + hconst[:, 0:1]
    val = _colsum(heads[:, 1:2]) * inv_n + hconst[:, 1:2]
    values_ref[0] = val


def kernel(states, W0, b0, W1, b1, W2, b2, Wa, ba, Wc, bc):
    x = states.reshape(B, N_NODES, D_IN)
    b0r = b0.reshape(1, HID)
    b1r = b1.reshape(1, HID)
    b2r = b2.reshape(1, HID)
    wht = jnp.concatenate([Wa[:HID, :], Wc[:HID, :]], axis=1)
    whb = jnp.concatenate([Wa[HID:, :], Wc[HID:, :]], axis=1)
    bh = jnp.concatenate([ba, bc]).reshape(1, 2)

    full = lambda shape: pl.BlockSpec(shape, lambda i: (0,) * len(shape))

    logits, values = pl.pallas_call(
        _fused_kernel,
        grid=(B,),
        in_specs=[
            pl.BlockSpec((1, N_NODES, D_IN), lambda i: (i, 0, 0)),
            full((2 * D_IN, HID)), full((1, HID)),
            full((2 * HID, HID)), full((1, HID)),
            full((2 * HID, HID)), full((1, HID)),
            full((HID, 2)), full((HID, 2)), full((1, 2)),
        ],
        out_specs=[
            pl.BlockSpec((1, N_NODES, 1), lambda i: (i, 0, 0)),
            pl.BlockSpec((1, 1, 1), lambda i: (i, 0, 0)),
        ],
        out_shape=[
            jax.ShapeDtypeStruct((B, N_NODES, 1), jnp.float32),
            jax.ShapeDtypeStruct((B, 1, 1), jnp.float32),
        ],
        compiler_params=pltpu.CompilerParams(
            dimension_semantics=("arbitrary",),
        ),
    )(x, W0, b0r, W1, b1r, W2, b2r, wht, whb, bh)

    return (logits.reshape(SEQ, ENVS, N_NODES), values.reshape(SEQ, ENVS))


# speedup vs baseline: 3.0612x; 1.0004x over previous
import jax
import jax.numpy as jnp
from jax.experimental import pallas as pl
from jax.experimental.pallas import tpu as pltpu

SEQ, ENVS, N_NODES, D_IN = 4, 8, 10000, 128
HID = 128
B = SEQ * ENVS


def _colsum(x):
    part = jnp.sum(x.reshape(10, N_NODES // 10, x.shape[-1]), axis=0)
    return jnp.sum(part, axis=0, keepdims=True)


def _fused_kernel(x_ref, w0_ref, b0_ref, w1_ref, b1_ref, w2_ref, b2_ref,
                  wht_ref, whb_ref, bh_ref, logits_ref, values_ref):
    x = x_ref[0]
    inv_n = jnp.float32(1.0 / N_NODES)

    def layer(x, w_ref, b_ref, skip=None):
        m = _colsum(x) * inv_n
        c = jnp.dot(m, w_ref[HID:, :],
                    preferred_element_type=jnp.float32) + b_ref[:]
        out = jnp.dot(x, w_ref[:HID, :],
                      preferred_element_type=jnp.float32) + c
        out = jnp.maximum(out, 0.0)
        if skip is not None:
            out = out + skip
        return out

    h0 = layer(x, w0_ref, b0_ref)
    h1 = layer(h0, w1_ref, b1_ref, skip=h0)
    h2 = layer(h1, w2_ref, b2_ref, skip=h1)

    m = _colsum(h2) * inv_n
    heads = jnp.dot(h2, wht_ref[:],
                    preferred_element_type=jnp.float32)
    hconst = jnp.dot(m, whb_ref[:],
                     preferred_element_type=jnp.float32) + bh_ref[:]

    logits_ref[0] = heads[:, 0:1] + hconst[:, 0:1]
    val = _colsum(heads[:, 1:2]) * inv_n + hconst[:, 1:2]
    values_ref[0] = val


def kernel(states, W0, b0, W1, b1, W2, b2, Wa, ba, Wc, bc):
    x = states.reshape(B, N_NODES, D_IN)
    b0r = b0.reshape(1, HID)
    b1r = b1.reshape(1, HID)
    b2r = b2.reshape(1, HID)
    wht = jnp.concatenate([Wa[:HID, :], Wc[:HID, :]], axis=1)
    whb = jnp.concatenate([Wa[HID:, :], Wc[HID:, :]], axis=1)
    bh = jnp.concatenate([ba, bc]).reshape(1, 2)

    full = lambda shape: pl.BlockSpec(shape, lambda i: (0,) * len(shape))

    logits, values = pl.pallas_call(
        _fused_kernel,
        grid=(B,),
        in_specs=[
            pl.BlockSpec((1, N_NODES, D_IN), lambda i: (i, 0, 0)),
            full((2 * D_IN, HID)), full((1, HID)),
            full((2 * HID, HID)), full((1, HID)),
            full((2 * HID, HID)), full((1, HID)),
            full((HID, 2)), full((HID, 2)), full((1, 2)),
        ],
        out_specs=[
            pl.BlockSpec((1, N_NODES, 1), lambda i: (i, 0, 0)),
            pl.BlockSpec((1, 1, 1), lambda i: (i, 0, 0)),
        ],
        out_shape=[
            jax.ShapeDtypeStruct((B, N_NODES, 1), jnp.float32),
            jax.ShapeDtypeStruct((B, 1, 1), jnp.float32),
        ],
        compiler_params=pltpu.CompilerParams(
            dimension_semantics=("parallel",),
        ),
    )(x, W0, b0r, W1, b1r, W2, b2r, wht, whb, bh)

    return (logits.reshape(SEQ, ENVS, N_NODES), values.reshape(SEQ, ENVS))


# speedup vs baseline: 3.0641x; 1.0009x over previous
import jax
import jax.numpy as jnp
from jax.experimental import pallas as pl
from jax.experimental.pallas import tpu as pltpu

SEQ, ENVS, N_NODES, D_IN = 4, 8, 10000, 128
HID = 128
B = SEQ * ENVS


def _colsum(x):
    part = jnp.sum(x.reshape(10, N_NODES // 10, x.shape[-1]), axis=0)
    return jnp.sum(part, axis=0, keepdims=True)


def _fused_kernel(x_ref, w0_ref, b0_ref, w1_ref, b1_ref, w2_ref, b2_ref,
                  wht_ref, whb_ref, bh_ref, logits_ref, values_ref):
    x = x_ref[0, 0]
    inv_n = jnp.float32(1.0 / N_NODES)

    def layer(x, w_ref, b_ref, skip=None):
        m = _colsum(x) * inv_n
        c = jnp.dot(m, w_ref[HID:, :],
                    preferred_element_type=jnp.float32) + b_ref[:]
        out = jnp.dot(x, w_ref[:HID, :],
                      preferred_element_type=jnp.float32) + c
        out = jnp.maximum(out, 0.0)
        if skip is not None:
            out = out + skip
        return out

    h0 = layer(x, w0_ref, b0_ref)
    h1 = layer(h0, w1_ref, b1_ref, skip=h0)
    h2 = layer(h1, w2_ref, b2_ref, skip=h1)

    m = _colsum(h2) * inv_n
    heads = jnp.dot(h2, wht_ref[:],
                    preferred_element_type=jnp.float32)
    hconst = jnp.dot(m, whb_ref[:],
                     preferred_element_type=jnp.float32) + bh_ref[:]

    logits_ref[0, 0] = heads[:, 0:1] + hconst[:, 0:1]
    val = _colsum(heads[:, 1:2]) * inv_n + hconst[:, 1:2]
    values_ref[0, 0] = val


def kernel(states, W0, b0, W1, b1, W2, b2, Wa, ba, Wc, bc):
    b0r = b0.reshape(1, HID)
    b1r = b1.reshape(1, HID)
    b2r = b2.reshape(1, HID)
    wht = jnp.concatenate([Wa[:HID, :], Wc[:HID, :]], axis=1)
    whb = jnp.concatenate([Wa[HID:, :], Wc[HID:, :]], axis=1)
    bh = jnp.concatenate([ba, bc]).reshape(1, 2)

    full = lambda shape: pl.BlockSpec(shape, lambda s, e: (0,) * len(shape))

    logits, values = pl.pallas_call(
        _fused_kernel,
        grid=(SEQ, ENVS),
        in_specs=[
            pl.BlockSpec((1, 1, N_NODES, D_IN), lambda s, e: (s, e, 0, 0)),
            full((2 * D_IN, HID)), full((1, HID)),
            full((2 * HID, HID)), full((1, HID)),
            full((2 * HID, HID)), full((1, HID)),
            full((HID, 2)), full((HID, 2)), full((1, 2)),
        ],
        out_specs=[
            pl.BlockSpec((1, 1, N_NODES, 1), lambda s, e: (s, e, 0, 0)),
            pl.BlockSpec((1, 1, 1, 1), lambda s, e: (s, e, 0, 0)),
        ],
        out_shape=[
            jax.ShapeDtypeStruct((SEQ, ENVS, N_NODES, 1), jnp.float32),
            jax.ShapeDtypeStruct((SEQ, ENVS, 1, 1), jnp.float32),
        ],
        compiler_params=pltpu.CompilerParams(
            dimension_semantics=("parallel", "parallel"),
        ),
    )(states, W0, b0r, W1, b1r, W2, b2r, wht, whb, bh)

    return (logits.reshape(SEQ, ENVS, N_NODES), values.reshape(SEQ, ENVS))


# speedup vs baseline: 3.0959x; 1.0104x over previous
import jax
import jax.numpy as jnp
from jax.experimental import pallas as pl
from jax.experimental.pallas import tpu as pltpu

SEQ, ENVS, N_NODES, D_IN = 4, 8, 10000, 128
HID = 128
B = SEQ * ENVS


def _colsum(x):
    part = jnp.sum(x.reshape(10, N_NODES // 10, x.shape[-1]), axis=0)
    return jnp.sum(part, axis=0, keepdims=True)


def _fused_kernel(x_ref, w0_ref, b0_ref, w1_ref, b1_ref, w2_ref, b2_ref,
                  wht_ref, whb_ref, bh_ref, logits_ref, values_ref):
    x = x_ref[0, 0]
    inv_n = jnp.float32(1.0 / N_NODES)

    def layer(x, w_ref, b_ref, skip=None):
        m = _colsum(x) * inv_n
        c = jnp.dot(m, w_ref[HID:, :],
                    preferred_element_type=jnp.float32) + b_ref[:]
        out = jnp.dot(x, w_ref[:HID, :],
                      preferred_element_type=jnp.float32) + c
        out = jnp.maximum(out, 0.0)
        if skip is not None:
            out = out + skip
        return out

    h0 = layer(x, w0_ref, b0_ref)
    h1 = layer(h0, w1_ref, b1_ref, skip=h0)
    h2 = layer(h1, w2_ref, b2_ref, skip=h1)

    m = _colsum(h2) * inv_n
    heads = jnp.dot(h2, wht_ref[:],
                    preferred_element_type=jnp.float32)
    hconst = jnp.dot(m, whb_ref[:],
                     preferred_element_type=jnp.float32) + bh_ref[:]

    logits_ref[0, 0] = (heads[:, 0:1] + hconst[:, 0:1]).reshape(N_NODES // 8, 8)
    val = _colsum(heads[:, 1:2]) * inv_n + hconst[:, 1:2]
    values_ref[0, 0] = val


def kernel(states, W0, b0, W1, b1, W2, b2, Wa, ba, Wc, bc):
    b0r = b0.reshape(1, HID)
    b1r = b1.reshape(1, HID)
    b2r = b2.reshape(1, HID)
    wht = jnp.concatenate([Wa[:HID, :], Wc[:HID, :]], axis=1)
    whb = jnp.concatenate([Wa[HID:, :], Wc[HID:, :]], axis=1)
    bh = jnp.concatenate([ba, bc]).reshape(1, 2)

    full = lambda shape: pl.BlockSpec(shape, lambda s, e: (0,) * len(shape))

    logits, values = pl.pallas_call(
        _fused_kernel,
        grid=(SEQ, ENVS),
        in_specs=[
            pl.BlockSpec((1, 1, N_NODES, D_IN), lambda s, e: (s, e, 0, 0)),
            full((2 * D_IN, HID)), full((1, HID)),
            full((2 * HID, HID)), full((1, HID)),
            full((2 * HID, HID)), full((1, HID)),
            full((HID, 2)), full((HID, 2)), full((1, 2)),
        ],
        out_specs=[
            pl.BlockSpec((1, 1, N_NODES // 8, 8), lambda s, e: (s, e, 0, 0)),
            pl.BlockSpec((1, 1, 1, 1), lambda s, e: (s, e, 0, 0)),
        ],
        out_shape=[
            jax.ShapeDtypeStruct((SEQ, ENVS, N_NODES // 8, 8), jnp.float32),
            jax.ShapeDtypeStruct((SEQ, ENVS, 1, 1), jnp.float32),
        ],
        compiler_params=pltpu.CompilerParams(
            dimension_semantics=("parallel", "parallel"),
        ),
    )(states, W0, b0r, W1, b1r, W2, b2r, wht, whb, bh)

    return (logits.reshape(SEQ, ENVS, N_NODES), values.reshape(SEQ, ENVS))


# speedup vs baseline: 3.1973x; 1.0327x over previous
import jax
import jax.numpy as jnp
from jax.experimental import pallas as pl
from jax.experimental.pallas import tpu as pltpu

SEQ, ENVS, N_NODES, D_IN = 4, 8, 10000, 128
HID = 128
B = SEQ * ENVS


def _colsum(x):
    part = jnp.sum(x.reshape(10, N_NODES // 10, x.shape[-1]), axis=0)
    return jnp.sum(part, axis=0, keepdims=True)


def _colsum2(x):
    part = jnp.sum(x.reshape(2, 10, N_NODES // 10, x.shape[-1]), axis=1)
    return jnp.sum(part, axis=1, keepdims=True)


def _fused_kernel(x_ref, w0_ref, b0_ref, w1_ref, b1_ref, w2_ref, b2_ref,
                  wht_ref, whb_ref, bh_ref, logits_ref, values_ref):
    x = x_ref[0]
    inv_n = jnp.float32(1.0 / N_NODES)

    def layer(x, w_ref, b_ref, skip=None):
        m = _colsum2(x) * inv_n
        c = jnp.matmul(m, w_ref[HID:, :],
                       preferred_element_type=jnp.float32) + b_ref[:]
        out = jnp.matmul(x, w_ref[:HID, :],
                         preferred_element_type=jnp.float32) + c
        out = jnp.maximum(out, 0.0)
        if skip is not None:
            out = out + skip
        return out

    h0 = layer(x, w0_ref, b0_ref)
    h1 = layer(h0, w1_ref, b1_ref, skip=h0)
    h2 = layer(h1, w2_ref, b2_ref, skip=h1)

    m = _colsum2(h2) * inv_n
    heads = jnp.matmul(h2, wht_ref[:],
                       preferred_element_type=jnp.float32)
    hconst = jnp.matmul(m, whb_ref[:],
                        preferred_element_type=jnp.float32) + bh_ref[:]

    logits = heads[:, :, 0:1] + hconst[:, :, 0:1]
    logits_ref[0] = logits.reshape(2, N_NODES // 8, 8)
    val = _colsum2(heads[:, :, 1:2]) * inv_n + hconst[:, :, 1:2]
    values_ref[0] = val.reshape(2, 1, 1)


def kernel(states, W0, b0, W1, b1, W2, b2, Wa, ba, Wc, bc):
    b0r = b0.reshape(1, HID)
    b1r = b1.reshape(1, HID)
    b2r = b2.reshape(1, HID)
    wht = jnp.concatenate([Wa[:HID, :], Wc[:HID, :]], axis=1)
    whb = jnp.concatenate([Wa[HID:, :], Wc[HID:, :]], axis=1)
    bh = jnp.concatenate([ba, bc]).reshape(1, 2)

    full = lambda shape: pl.BlockSpec(shape, lambda s, e: (0,) * len(shape))

    xb = states.reshape(B // 2, 2, N_NODES, D_IN)
    fullb = lambda shape: pl.BlockSpec(shape, lambda i: (0,) * len(shape))

    logits, values = pl.pallas_call(
        _fused_kernel,
        grid=(B // 2,),
        in_specs=[
            pl.BlockSpec((1, 2, N_NODES, D_IN), lambda i: (i, 0, 0, 0)),
            fullb((2 * D_IN, HID)), fullb((1, HID)),
            fullb((2 * HID, HID)), fullb((1, HID)),
            fullb((2 * HID, HID)), fullb((1, HID)),
            fullb((HID, 2)), fullb((HID, 2)), fullb((1, 2)),
        ],
        out_specs=[
            pl.BlockSpec((1, 2, N_NODES // 8, 8), lambda i: (i, 0, 0, 0)),
            pl.BlockSpec((1, 2, 1, 1), lambda i: (i, 0, 0, 0)),
        ],
        out_shape=[
            jax.ShapeDtypeStruct((B // 2, 2, N_NODES // 8, 8), jnp.float32),
            jax.ShapeDtypeStruct((B // 2, 2, 1, 1), jnp.float32),
        ],
        compiler_params=pltpu.CompilerParams(
            dimension_semantics=("parallel",),
        ),
    )(xb, W0, b0r, W1, b1r, W2, b2r, wht, whb, bh)

    return (logits.reshape(SEQ, ENVS, N_NODES), values.reshape(SEQ, ENVS))
